# baseline (speedup 1.0000x reference)
import jax
import jax.numpy as jnp
from jax import lax
from jax.experimental import pallas as pl
from jax.experimental.pallas import tpu as pltpu

_B, _H, _W, _K = 4, 48, 48, 9
_N = _H * _W * _K
_S = 8
_C = _N // _S
_MOS = 100
_IOU_T = 0.9
_SCORE_T = 0.9
_PROP_T = 0.5
_NEG_INF = float("-inf")


_BG = 2


def _nms_kernel(score_ref, delta_ref, anchor_ref, out_ref):
    tx = delta_ref[0]
    ty = delta_ref[1]
    tw = delta_ref[2]
    th = delta_ref[3]
    a0 = anchor_ref[0:1, :, :]
    a1 = anchor_ref[1:2, :, :]
    a2 = anchor_ref[2:3, :, :]
    a3 = anchor_ref[3:4, :, :]
    xa = (a0 + a1) * 0.5
    ya = (a2 + a3) * 0.5
    wa = a1 - a0
    ha = a3 - a2
    x = tx * wa + xa
    y = ty * ha + ya
    w = jnp.exp(tw) * wa
    h = jnp.exp(th) * ha
    o_ymax = jnp.minimum(y + h * 0.5, 1.0)
    o_xmin = jnp.maximum(x - w * 0.5, 0.0)
    o_ymin = jnp.maximum(y - h * 0.5, 0.0)
    o_xmax = jnp.minimum(x + w * 0.5, 1.0)
    ymin = jnp.minimum(o_ymin, o_ymax)
    ymax = jnp.maximum(o_ymin, o_ymax)
    xmin = jnp.minimum(o_xmin, o_xmax)
    xmax = jnp.maximum(o_xmin, o_xmax)
    area = (ymax - ymin) * (xmax - xmin)

    sc0 = jnp.where(score_ref[...] > _PROP_T, score_ref[...], _NEG_INF)
    iota = (lax.broadcasted_iota(jnp.int32, (_BG, _S, _C), 1) * _C
            + lax.broadcasted_iota(jnp.int32, (_BG, _S, _C), 2))

    def body(i, sc):
        best_val = jnp.max(sc, axis=(1, 2), keepdims=True)
        eq = sc == best_val
        bidx = jnp.min(jnp.where(eq, iota, _N), axis=(1, 2), keepdims=True)
        onehot = iota == bidx

        def sel(v):
            return jnp.sum(jnp.where(onehot, v, 0.0), axis=(1, 2), keepdims=True)

        b_oymax = sel(o_ymax)
        b_oxmin = sel(o_xmin)
        b_oymin = sel(o_ymin)
        b_oxmax = sel(o_xmax)
        b_ymin = jnp.minimum(b_oymin, b_oymax)
        b_ymax = jnp.maximum(b_oymin, b_oymax)
        b_xmin = jnp.minimum(b_oxmin, b_oxmax)
        b_xmax = jnp.maximum(b_oxmin, b_oxmax)
        b_area = (b_ymax - b_ymin) * (b_xmax - b_xmin)

        valid = (best_val > _SCORE_T).astype(jnp.float32)
        row = jnp.concatenate(
            [b_oymax * valid, b_oxmin * valid, b_oymin * valid, b_oxmax * valid],
            axis=2,
        )
        for b in range(_BG):
            out_ref[:, pl.ds(i, 1), 4 * b:4 * b + 4] = row[b:b + 1]

        iy1 = jnp.maximum(b_ymin, ymin)
        iy2 = jnp.minimum(b_ymax, ymax)
        ix1 = jnp.maximum(b_xmin, xmin)
        ix2 = jnp.minimum(b_xmax, xmax)
        inter = jnp.maximum(iy2 - iy1, 0.0) * jnp.maximum(ix2 - ix1, 0.0)
        iou = inter / (b_area + area - inter + 1e-8)
        sc = jnp.where(jnp.logical_or(iou > _IOU_T, onehot), _NEG_INF, sc)
        return sc

    lax.fori_loop(0, _MOS, body, sc0)


def kernel(score_map, delta_map, anchors):
    scores = score_map.reshape(_B, _S, _C)
    deltas = delta_map.reshape(_B, _N, 4).transpose(2, 0, 1).reshape(4, _B, _S, _C)
    anc = anchors.reshape(_N, 4).T.reshape(4, _S, _C)
    out = pl.pallas_call(
        _nms_kernel,
        grid=(_B // _BG,),
        in_specs=[
            pl.BlockSpec((_BG, _S, _C), lambda i: (i, 0, 0)),
            pl.BlockSpec((4, _BG, _S, _C), lambda i: (0, i, 0, 0)),
            pl.BlockSpec((4, _S, _C), lambda i: (0, 0, 0)),
        ],
        out_specs=pl.BlockSpec((1, _MOS, _BG * 4), lambda i: (i, 0, 0)),
        out_shape=jax.ShapeDtypeStruct((_B // _BG, _MOS, _BG * 4), jnp.float32),
        compiler_params=pltpu.CompilerParams(dimension_semantics=("parallel",)),
    )(scores, deltas, anc)
    return out.reshape(_B // _BG, _MOS, _BG, 4).transpose(0, 2, 1, 3).reshape(_B, _MOS, 4)


# speedup vs baseline: 1.0509x; 1.0509x over previous
import jax
import jax.numpy as jnp
from jax import lax
from jax.experimental import pallas as pl
from jax.experimental.pallas import tpu as pltpu

_B, _H, _W, _K = 4, 48, 48, 9
_N = _H * _W * _K
_S = 8
_C = _N // _S
_MOS = 100
_IOU_T = 0.9
_SCORE_T = 0.9
_NEG_INF = float("-inf")


def _nms_kernel(score_ref, delta_ref, anchor_ref, out_ref, sc_ref, acc_ref, cd_ref):
    tx = delta_ref[0]
    ty = delta_ref[1]
    tw = delta_ref[2]
    th = delta_ref[3]
    a0 = anchor_ref[0:1, :, :]
    a1 = anchor_ref[1:2, :, :]
    a2 = anchor_ref[2:3, :, :]
    a3 = anchor_ref[3:4, :, :]
    xa = (a0 + a1) * 0.5
    ya = (a2 + a3) * 0.5
    wa = a1 - a0
    ha = a3 - a2
    x = tx * wa + xa
    y = ty * ha + ya
    w = jnp.exp(tw) * wa
    h = jnp.exp(th) * ha
    o_ymax = jnp.minimum(y + h * 0.5, 1.0)
    o_xmin = jnp.maximum(x - w * 0.5, 0.0)
    o_ymin = jnp.maximum(y - h * 0.5, 0.0)
    o_xmax = jnp.minimum(x + w * 0.5, 1.0)

    iota = (lax.broadcasted_iota(jnp.int32, (_B, _S, _C), 1) * _C
            + lax.broadcasted_iota(jnp.int32, (_B, _S, _C), 2))
    lane128 = lax.broadcasted_iota(jnp.int32, (_B, 1, 128), 2)
    row_iota = lax.broadcasted_iota(jnp.int32, (_B, _MOS, 4), 1)

    sc_ref[...] = jnp.where(score_ref[...] > _SCORE_T, score_ref[...], _NEG_INF)
    acc_ref[...] = jnp.zeros((_B, 8, 128), jnp.float32)
    cd_ref[...] = jnp.zeros((_B, 8, 128), jnp.int32)
    out_ref[...] = jnp.zeros((_B, _MOS, 4), jnp.float32)

    def cond(alldone):
        return jnp.logical_not(alldone)

    def body(alldone):
        sc = sc_ref[...]
        best_val = jnp.max(sc, axis=(1, 2), keepdims=True)
        eq = sc == best_val
        bidx = jnp.min(jnp.where(eq, iota, _N), axis=(1, 2), keepdims=True)
        onehot = iota == bidx
        sc_ref[...] = jnp.where(onehot, _NEG_INF, sc)

        def sel(v):
            return jnp.sum(jnp.where(onehot, v, 0.0), axis=(1, 2), keepdims=True)

        c_oymax = sel(o_ymax)
        c_oxmin = sel(o_xmin)
        c_oymin = sel(o_ymin)
        c_oxmax = sel(o_xmax)
        c_ymin = jnp.minimum(c_oymin, c_oymax)
        c_ymax = jnp.maximum(c_oymin, c_oymax)
        c_xmin = jnp.minimum(c_oxmin, c_oxmax)
        c_xmax = jnp.maximum(c_oxmin, c_oxmax)
        c_area = (c_ymax - c_ymin) * (c_xmax - c_xmin)

        aymin = acc_ref[:, 0:1, :]
        aymax = acc_ref[:, 1:2, :]
        axmin = acc_ref[:, 2:3, :]
        axmax = acc_ref[:, 3:4, :]
        aarea = acc_ref[:, 4:5, :]
        cnt = cd_ref[:, 0:1, :]
        done = cd_ref[:, 1:2, 0:1] > 0

        done = jnp.logical_or(done, best_val <= _SCORE_T)

        iy1 = jnp.maximum(aymin, c_ymin)
        iy2 = jnp.minimum(aymax, c_ymax)
        ix1 = jnp.maximum(axmin, c_xmin)
        ix2 = jnp.minimum(axmax, c_xmax)
        inter = jnp.maximum(iy2 - iy1, 0.0) * jnp.maximum(ix2 - ix1, 0.0)
        iou = inter / (aarea + c_area - inter + 1e-8)
        hit = jnp.logical_and(iou > _IOU_T, lane128 < cnt)
        suppressed = jnp.any(hit, axis=2, keepdims=True)

        accept = jnp.logical_and(jnp.logical_not(done),
                                 jnp.logical_not(suppressed))
        slot = lane128 == cnt
        app = jnp.logical_and(slot, accept)
        acc_ref[:, 0:1, :] = jnp.where(app, c_ymin, aymin)
        acc_ref[:, 1:2, :] = jnp.where(app, c_ymax, aymax)
        acc_ref[:, 2:3, :] = jnp.where(app, c_xmin, axmin)
        acc_ref[:, 3:4, :] = jnp.where(app, c_xmax, axmax)
        acc_ref[:, 4:5, :] = jnp.where(app, c_area, aarea)

        cand_row = jnp.concatenate([c_oymax, c_oxmin, c_oymin, c_oxmax],
                                   axis=2)
        wmask = jnp.logical_and(row_iota == cnt[:, :, 0:1], accept)
        out_ref[...] = jnp.where(wmask, cand_row, out_ref[...])

        cnt = cnt + accept.astype(jnp.int32)
        done = jnp.logical_or(done, cnt[:, :, 0:1] >= _MOS)
        cd_ref[:, 0:1, :] = cnt
        cd_ref[:, 1:2, :] = jnp.broadcast_to(done, (_B, 1, 128)).astype(jnp.int32)
        return jnp.min(done.astype(jnp.int32)) == 1

    lax.while_loop(cond, body, jnp.bool_(False))


def kernel(score_map, delta_map, anchors):
    scores = score_map.reshape(_B, _S, _C)
    deltas = delta_map.reshape(_B, _N, 4).transpose(2, 0, 1).reshape(4, _B, _S, _C)
    anc = anchors.reshape(_N, 4).T.reshape(4, _S, _C)
    out = pl.pallas_call(
        _nms_kernel,
        out_shape=jax.ShapeDtypeStruct((_B, _MOS, 4), jnp.float32),
        scratch_shapes=[
            pltpu.VMEM((_B, _S, _C), jnp.float32),
            pltpu.VMEM((_B, 8, 128), jnp.float32),
            pltpu.VMEM((_B, 8, 128), jnp.int32),
        ],
    )(scores, deltas, anc)
    return out


# speedup vs baseline: 1.4735x; 1.4021x over previous
import functools

import jax
import jax.numpy as jnp
from jax import lax
from jax.experimental import pallas as pl
from jax.experimental.pallas import tpu as pltpu
from jax.experimental.pallas import tpu_sc as plsc

_B, _H, _W, _K = 4, 48, 48, 9
_N = _H * _W * _K
_S = 8
_C = _N // _S
_MOS = 100
_IOU_T = 0.9
_SCORE_T = 0.9
_NEG_INF = float("-inf")
_L = 16
_NV = _N // _L
_ACC_CAP = 128
_OUT_CAP = 512


def _decode_kernel(delta_ref, anchor_ref, box_ref):
    tx = delta_ref[0]
    ty = delta_ref[1]
    tw = delta_ref[2]
    th = delta_ref[3]
    a0 = anchor_ref[0:1, :, :]
    a1 = anchor_ref[1:2, :, :]
    a2 = anchor_ref[2:3, :, :]
    a3 = anchor_ref[3:4, :, :]
    xa = (a0 + a1) * 0.5
    ya = (a2 + a3) * 0.5
    wa = a1 - a0
    ha = a3 - a2
    x = tx * wa + xa
    y = ty * ha + ya
    w = jnp.exp(tw) * wa
    h = jnp.exp(th) * ha
    box_ref[0] = jnp.minimum(y + h * 0.5, 1.0)
    box_ref[1] = jnp.maximum(x - w * 0.5, 0.0)
    box_ref[2] = jnp.maximum(y - h * 0.5, 0.0)
    box_ref[3] = jnp.minimum(x + w * 0.5, 1.0)


def _nms_sc_kernel(score_hbm, box_hbm, out_hbm,
                   sco_v, idx_v, oym_v, oxm_v, oyn_v, oxn_v,
                   aym_v, ayx_v, axm_v, axx_v, aar_v, out_v, cmax_v):
    c = lax.axis_index("c")
    s = lax.axis_index("s")
    b = s

    @pl.when(jnp.logical_and(s < 4, c == 0))
    def _work():
        pltpu.sync_copy(score_hbm.at[b], sco_v.at[pl.ds(0, _N)])
        pltpu.sync_copy(box_hbm.at[0, b], oym_v.at[pl.ds(0, _N)])
        pltpu.sync_copy(box_hbm.at[1, b], oxm_v.at[pl.ds(0, _N)])
        pltpu.sync_copy(box_hbm.at[2, b], oyn_v.at[pl.ds(0, _N)])
        pltpu.sync_copy(box_hbm.at[3, b], oxn_v.at[pl.ds(0, _N)])

        lane = lax.iota(jnp.int32, _L)
        neg = jnp.full((_L,), _NEG_INF, jnp.float32)

        zero = jnp.zeros((_L,), jnp.float32)
        lane0 = lane == 0
        for j in range(_OUT_CAP // _L):
            out_v[pl.ds(j * _L, _L)] = zero

        def compact_body(i, cnt):
            base = i * (4 * _L)
            sc0 = sco_v[pl.ds(base, _L)]
            sc1 = sco_v[pl.ds(base + _L, _L)]
            sc2 = sco_v[pl.ds(base + 2 * _L, _L)]
            sc3 = sco_v[pl.ds(base + 3 * _L, _L)]
            m0 = sc0 > _SCORE_T
            m1 = sc1 > _SCORE_T
            m2 = sc2 > _SCORE_T
            m3 = sc3 > _SCORE_T
            pc0 = plsc.all_reduce_population_count(m0)[0]
            pc1 = plsc.all_reduce_population_count(m1)[0]
            pc2 = plsc.all_reduce_population_count(m2)[0]
            pc3 = plsc.all_reduce_population_count(m3)[0]
            c1 = cnt + pc0
            c2 = c1 + pc1
            c3 = c2 + pc2
            plsc.store_compressed(sco_v.at[pl.ds(cnt, _L)], sc0, mask=m0)
            plsc.store_compressed(idx_v.at[pl.ds(cnt, _L)], lane + base,
                                  mask=m0)
            plsc.store_compressed(sco_v.at[pl.ds(c1, _L)], sc1, mask=m1)
            plsc.store_compressed(idx_v.at[pl.ds(c1, _L)], lane + base + _L,
                                  mask=m1)
            plsc.store_compressed(sco_v.at[pl.ds(c2, _L)], sc2, mask=m2)
            plsc.store_compressed(idx_v.at[pl.ds(c2, _L)],
                                  lane + base + 2 * _L, mask=m2)
            plsc.store_compressed(sco_v.at[pl.ds(c3, _L)], sc3, mask=m3)
            plsc.store_compressed(idx_v.at[pl.ds(c3, _L)],
                                  lane + base + 3 * _L, mask=m3)
            return c3 + pc3

        cnt = lax.fori_loop(0, _NV // 4, compact_body, jnp.int32(0))
        sco_v[pl.ds(cnt, _L)] = neg
        nv = (cnt + _L - 1) // _L

        for j in range((_NV + _L) // _L):
            cmax_v[pl.ds(j * _L, _L)] = neg

        def cm_body(i, _):
            v = sco_v[pl.ds(i * _L, _L)]
            gmv = lax.reduce_max(v, (0,))
            plsc.store_compressed(cmax_v.at[pl.ds(i, _L)],
                                  jnp.zeros((_L,), jnp.float32) + gmv,
                                  mask=lane0)
            return jnp.int32(0)

        lax.fori_loop(0, nv, cm_body, jnp.int32(0))
        nvc = (nv + _L - 1) // _L

        def pick_body(state):
            n_acc, _ = state

            def amx_body(i, mi):
                m, im = mi
                v = cmax_v[pl.ds(i * _L, _L)]
                upd = v > m
                return (jnp.where(upd, v, m),
                        jnp.where(upd, lane + i * _L, im))

            m, im = lax.fori_loop(0, nvc, amx_body,
                                  (neg, jnp.zeros((_L,), jnp.int32)))
            gm = lax.reduce_max(m, (0,))
            have = gm > _NEG_INF
            q = lax.reduce_min(jnp.where(m == gm, im, _NV), (0,))

            def rescan_body(i, carry):
                v = sco_v[pl.ds(i * _L, _L)]
                lv = plsc.all_reduce_ffs(v == gm)[0]
                return (lv, v)

            l, chv = lax.fori_loop(q, q + 1, rescan_body,
                                   (jnp.int32(0), neg))
            p = q * _L + l

            def examine(n_acc):
                pidx = jnp.zeros((_L,), jnp.int32) + p
                oi = plsc.load_gather(idx_v, [pidx])
                oy_c = plsc.load_gather(oym_v, [oi])
                ox_c = plsc.load_gather(oxm_v, [oi])
                on_c = plsc.load_gather(oyn_v, [oi])
                ow_c = plsc.load_gather(oxn_v, [oi])
                c_ymin = jnp.minimum(on_c, oy_c)
                c_ymax = jnp.maximum(on_c, oy_c)
                c_xmin = jnp.minimum(ox_c, ow_c)
                c_xmax = jnp.maximum(ox_c, ow_c)
                c_area = (c_ymax - c_ymin) * (c_xmax - c_xmin)

                def iou_body(j, hit):
                    iy1 = jnp.maximum(aym_v[pl.ds(j * _L, _L)], c_ymin)
                    iy2 = jnp.minimum(ayx_v[pl.ds(j * _L, _L)], c_ymax)
                    ix1 = jnp.maximum(axm_v[pl.ds(j * _L, _L)], c_xmin)
                    ix2 = jnp.minimum(axx_v[pl.ds(j * _L, _L)], c_xmax)
                    inter = (jnp.maximum(iy2 - iy1, 0.0)
                             * jnp.maximum(ix2 - ix1, 0.0))
                    iou = inter / (aar_v[pl.ds(j * _L, _L)] + c_area
                                   - inter + 1e-8)
                    valid = (lane + j * _L) < n_acc
                    return jnp.logical_or(
                        hit, jnp.logical_and(iou > _IOU_T, valid))

                hit = lax.fori_loop(0, (n_acc + _L - 1) // _L, iou_body,
                                    jnp.zeros((_L,), jnp.bool_))
                suppressed = jnp.any(hit)
                accept = jnp.logical_and(have, jnp.logical_not(suppressed))

                @pl.when(accept)
                def _accept():
                    def put(ref, vec):
                        plsc.store_compressed(ref.at[pl.ds(n_acc, _L)],
                                              vec, mask=lane0)

                    put(aym_v, c_ymin)
                    put(ayx_v, c_ymax)
                    put(axm_v, c_xmin)
                    put(axx_v, c_xmax)
                    put(aar_v, c_area)
                    row = jnp.where(lane == 0, oy_c,
                          jnp.where(lane == 1, ox_c,
                          jnp.where(lane == 2, on_c, ow_c)))
                    plsc.store_compressed(out_v.at[pl.ds(4 * n_acc, _L)],
                                          row, mask=lane < 4)

                @pl.when(have)
                def _mask_examined():
                    plsc.store_compressed(sco_v.at[pl.ds(p, _L)], neg,
                                          mask=lane0)
                    chv2 = jnp.where(lane == l, neg, chv)
                    ngm = lax.reduce_max(chv2, (0,))
                    plsc.store_compressed(cmax_v.at[pl.ds(q, _L)],
                                          jnp.zeros((_L,), jnp.float32) + ngm,
                                          mask=lane0)

                return n_acc + accept.astype(jnp.int32)

            n_acc = examine(n_acc)
            return (n_acc, jnp.logical_and(have, n_acc < _MOS))

        lax.while_loop(lambda st: st[1], pick_body, (jnp.int32(0), cnt > 0))
        pltpu.sync_copy(out_v, out_hbm.at[b])


def kernel(score_map, delta_map, anchors):
    scores = score_map.reshape(_B, _N)
    deltas = delta_map.reshape(_B, _N, 4).transpose(2, 0, 1).reshape(4, _B, _S, _C)
    anc = anchors.reshape(_N, 4).T.reshape(4, _S, _C)
    boxes = pl.pallas_call(
        _decode_kernel,
        out_shape=jax.ShapeDtypeStruct((4, _B, _S, _C), jnp.float32),
    )(deltas, anc)
    boxes = boxes.reshape(4, _B, _N)

    mesh = plsc.VectorSubcoreMesh(core_axis_name="c", subcore_axis_name="s")
    nms = pl.kernel(
        _nms_sc_kernel, mesh=mesh,
        compiler_params=pltpu.CompilerParams(needs_layout_passes=False),
        out_type=jax.ShapeDtypeStruct((_B, _OUT_CAP), jnp.float32),
        scratch_types=[pltpu.VMEM((_N + _L,), jnp.float32),
                       pltpu.VMEM((_N + _L,), jnp.int32)]
        + [pltpu.VMEM((_N + _L,), jnp.float32)] * 4
        + [pltpu.VMEM((_ACC_CAP,), jnp.float32)] * 5
        + [pltpu.VMEM((_OUT_CAP,), jnp.float32),
           pltpu.VMEM((_NV + _L,), jnp.float32)],
    )
    out = nms(scores, boxes)
    return out[:, :_MOS * 4].reshape(_B, _MOS, 4)
